# baseline (speedup 1.0000x reference)
import jax
import jax.numpy as jnp
from jax import lax
from jax.experimental import pallas as pl
from jax.experimental.pallas import tpu as pltpu
from jax.experimental.pallas import tpu_sc as plsc

N_NODES = 10000
N_EDGES = 320000
D_NODE = 128
DE = 16
PK = 128 // DE
NR = N_EDGES // PK

NC = 2
NS = 16
NW = NC * NS
GC = 80
CKE = 2560
GPC = CKE // GC
NCK = N_EDGES // CKE
CPW = -(-NCK // NW)


def _proj_body(ns_ref, w_ref, b_ref, ps_ref, pd_ref):
    p = jnp.dot(ns_ref[...], w_ref[...], preferred_element_type=jnp.float32)
    p = p + b_ref[...]
    ps_ref[...] = p[:, 0:DE]
    pd_ref[...] = p[:, DE:2 * DE]


def _node_project(node_scalars, w_sd, b12):
    return pl.pallas_call(
        _proj_body,
        out_shape=[
            jax.ShapeDtypeStruct((N_NODES, DE), jnp.float32),
            jax.ShapeDtypeStruct((N_NODES, DE), jnp.float32),
        ],
    )(node_scalars, w_sd, b12)


def _gather_body(ps_hbm, pd_hbm, si_hbm, di_hbm, s_hbm,
                 si_v, di_v, rs_v, sem_s, sem_d):
    wid = lax.axis_index("s") * NC + lax.axis_index("c")
    for k in range(CPW):
        ck = wid + k * NW

        @pl.when(ck < NCK)
        def _():
            pltpu.sync_copy(si_hbm.at[ck], si_v)
            pltpu.sync_copy(di_hbm.at[ck], di_v)
            descs = []
            for j in range(GPC):
                descs.append(pltpu.async_copy(
                    ps_hbm.at[si_v.at[j]], rs_v.at[pl.ds(j * GC, GC)], sem_s))
            for dsc in descs:
                dsc.wait()
            descs = []
            for j in range(GPC):
                descs.append(pltpu.async_copy(
                    pd_hbm.at[di_v.at[j]], rs_v.at[pl.ds(j * GC, GC)], sem_d,
                    add=True))
            for dsc in descs:
                dsc.wait()
            pltpu.sync_copy(rs_v, s_hbm.at[pl.ds(ck * CKE, CKE)])


def _sc_gather(psrc, pdst, si3, di3):
    f = pl.kernel(
        _gather_body,
        out_type=jax.ShapeDtypeStruct((N_EDGES, DE), jnp.float32),
        mesh=plsc.VectorSubcoreMesh(core_axis_name="c", subcore_axis_name="s",
                                    num_cores=NC, num_subcores=NS),
        scratch_types=[
            pltpu.VMEM((GPC, GC), jnp.int32),
            pltpu.VMEM((GPC, GC), jnp.int32),
            pltpu.VMEM((CKE, DE), jnp.float32),
            pltpu.SemaphoreType.DMA,
            pltpu.SemaphoreType.DMA,
        ],
        compiler_params=pltpu.CompilerParams(use_tc_tiling_on_sc=False),
    )
    return f(psrc, pdst, si3, di3)


BR = 1600
NBR = NR // BR


def _epi_body(s, ef, dd, b1e, b1d, b2m, mm, b2r, gr, br, out):
    x = s[...]
    x = x + jnp.dot(ef[...], b1e[...], preferred_element_type=jnp.float32)
    x = x + jnp.dot(dd[...], b1d[...], preferred_element_type=jnp.float32)
    h1 = x * jax.nn.sigmoid(x)
    y = jnp.dot(h1, b2m[...], preferred_element_type=jnp.float32)
    y = y + b2r[...][0:1, :]
    h2 = y * jax.nn.sigmoid(y)
    r = ef[...] + h2
    m = jnp.dot(r, mm[...], preferred_element_type=jnp.float32)
    q = r - m
    v = jnp.dot(q * q, mm[...], preferred_element_type=jnp.float32)
    out[...] = q * lax.rsqrt(v + 1e-5) * gr[...][0:1, :] + br[...][0:1, :]


def _epilogue(s4, ef4, d4, b1e, b1d, b2m, mm, b2r, gr, br):
    big = pl.BlockSpec((BR, 128), lambda i: (i, 0))
    wsp = pl.BlockSpec((128, 128), lambda i: (0, 0))
    vsp = pl.BlockSpec((8, 128), lambda i: (0, 0))
    return pl.pallas_call(
        _epi_body,
        grid=(NBR,),
        in_specs=[big, big, big, wsp, wsp, wsp, wsp, vsp, vsp, vsp],
        out_specs=big,
        out_shape=jax.ShapeDtypeStruct((NR, 128), jnp.float32),
    )(s4, ef4, d4, b1e, b1d, b2m, mm, b2r, gr, br)


def _blockdiag8(w):
    eye8 = jnp.eye(PK, dtype=w.dtype)
    return jnp.einsum("ab,ij->aibj", eye8, w).reshape(128, 128)


def kernel(node_scalars, edge_feats, d, src_idxs, dst_idxs,
           W1, b1, W2, b2, ln_g, ln_b):
    w_sd = jnp.concatenate([W1[0:D_NODE], W1[D_NODE:2 * D_NODE]], axis=1)
    b12 = 0.5 * jnp.concatenate([b1, b1])[None, :]
    b1e = _blockdiag8(W1[2 * D_NODE:2 * D_NODE + DE])
    b1d = _blockdiag8(W1[2 * D_NODE + DE:2 * D_NODE + 2 * DE])
    b2m = _blockdiag8(W2)
    mm = _blockdiag8(jnp.full((DE, DE), 1.0 / DE, dtype=jnp.float32))
    b2r = jnp.tile(jnp.tile(b2, PK)[None, :], (8, 1))
    gr = jnp.tile(jnp.tile(ln_g, PK)[None, :], (8, 1))
    br = jnp.tile(jnp.tile(ln_b, PK)[None, :], (8, 1))

    psrc, pdst = _node_project(node_scalars, w_sd, b12)

    si3 = src_idxs.astype(jnp.int32).reshape(NCK, GPC, GC)
    di3 = dst_idxs.astype(jnp.int32).reshape(NCK, GPC, GC)
    s = _sc_gather(psrc, pdst, si3, di3)

    out4 = _epilogue(s.reshape(NR, 128), edge_feats.reshape(NR, 128),
                     d.reshape(NR, 128), b1e, b1d, b2m, mm, b2r, gr, br)
    return out4.reshape(N_EDGES, DE)


# speedup vs baseline: 1.9150x; 1.9150x over previous
import jax
import jax.numpy as jnp
from jax import lax
from jax.experimental import pallas as pl
from jax.experimental.pallas import tpu as pltpu
from jax.experimental.pallas import tpu_sc as plsc

N_NODES = 10000
N_EDGES = 320000
D_NODE = 128
DE = 16

NC = 2
NS = 16
NW = NC * NS
GC = 80
CKE = 2560
GPC = CKE // GC
NCK = N_EDGES // CKE
CPW = -(-NCK // NW)


def _proj_body(ns_ref, w_ref, b_ref, ps_ref, pd_ref):
    p = jnp.dot(ns_ref[...], w_ref[...], preferred_element_type=jnp.float32)
    p = p + b_ref[...]
    ps_ref[...] = p[:, 0:DE]
    pd_ref[...] = p[:, DE:2 * DE]


def _node_project(node_scalars, w_sd, b12):
    return pl.pallas_call(
        _proj_body,
        out_shape=[
            jax.ShapeDtypeStruct((N_NODES, DE), jnp.float32),
            jax.ShapeDtypeStruct((N_NODES, DE), jnp.float32),
        ],
    )(node_scalars, w_sd, b12)


def _gather_body(nck, ps_hbm, pd_hbm, si_hbm, di_hbm, s_hbm,
                 si_v, di_v, rs_v, sem_s, sem_d):
    wid = lax.axis_index("s") * NC + lax.axis_index("c")
    for k in range(-(-nck // NW)):
        ck = wid + k * NW

        @pl.when(ck < nck)
        def _():
            pltpu.sync_copy(si_hbm.at[ck], si_v)
            pltpu.sync_copy(di_hbm.at[ck], di_v)
            descs = []
            for j in range(GPC):
                descs.append(pltpu.async_copy(
                    ps_hbm.at[si_v.at[j]], rs_v.at[pl.ds(j * GC, GC)], sem_s))
            for dsc in descs:
                dsc.wait()
            descs = []
            for j in range(GPC):
                descs.append(pltpu.async_copy(
                    pd_hbm.at[di_v.at[j]], rs_v.at[pl.ds(j * GC, GC)], sem_d,
                    add=True))
            for dsc in descs:
                dsc.wait()
            pltpu.sync_copy(rs_v, s_hbm.at[pl.ds(ck * CKE, CKE)])


def _sc_gather(psrc, pdst, si3, di3):
    nck = si3.shape[0]
    f = pl.kernel(
        lambda *refs: _gather_body(nck, *refs),
        out_type=jax.ShapeDtypeStruct((nck * CKE, DE), jnp.float32),
        mesh=plsc.VectorSubcoreMesh(core_axis_name="c", subcore_axis_name="s",
                                    num_cores=NC, num_subcores=NS),
        scratch_types=[
            pltpu.VMEM((GPC, GC), jnp.int32),
            pltpu.VMEM((GPC, GC), jnp.int32),
            pltpu.VMEM((CKE, DE), jnp.float32),
            pltpu.SemaphoreType.DMA,
            pltpu.SemaphoreType.DMA,
        ],
        compiler_params=pltpu.CompilerParams(use_tc_tiling_on_sc=False),
    )
    return f(psrc, pdst, si3, di3)


BL = 12800
NBL = N_EDGES // BL


NBA = 13
NCK_A = NBA * BL // CKE


def _epi_body(sa, sb, ef, dd, w1et, w1dt, w2t, b2c, gc, bc, out):
    i = pl.program_id(0)
    x = jnp.where(i < NBA, sa[...], sb[...])
    x = x + jnp.dot(w1et[...], ef[...], preferred_element_type=jnp.float32)
    x = x + jnp.dot(w1dt[...], dd[...], preferred_element_type=jnp.float32)
    h1 = x * jax.nn.sigmoid(x)
    y = jnp.dot(w2t[...], h1, preferred_element_type=jnp.float32)
    y = y + b2c[...][:, 0:1]
    h2 = y * jax.nn.sigmoid(y)
    r = ef[...] + h2
    m = jnp.mean(r, axis=0, keepdims=True)
    q = r - m
    v = jnp.mean(q * q, axis=0, keepdims=True)
    out[...] = q * lax.rsqrt(v + 1e-5) * gc[...][:, 0:1] + bc[...][:, 0:1]


def _epilogue(sta, stb, eft, dt, w1et, w1dt, w2t, b2c, gc, bc):
    big = pl.BlockSpec((DE, BL), lambda i: (0, i))
    spa = pl.BlockSpec((DE, BL), lambda i: (0, lax.min(i, NBA - 1)))
    spb = pl.BlockSpec((DE, BL), lambda i: (0, lax.max(i - NBA, 0)))
    wsp = pl.BlockSpec((DE, DE), lambda i: (0, 0))
    vsp = pl.BlockSpec((DE, 128), lambda i: (0, 0))
    return pl.pallas_call(
        _epi_body,
        grid=(NBL,),
        in_specs=[spa, spb, big, big, wsp, wsp, wsp, vsp, vsp, vsp],
        out_specs=big,
        out_shape=jax.ShapeDtypeStruct((DE, N_EDGES), jnp.float32),
    )(sta, stb, eft, dt, w1et, w1dt, w2t, b2c, gc, bc)


def kernel(node_scalars, edge_feats, d, src_idxs, dst_idxs,
           W1, b1, W2, b2, ln_g, ln_b):
    w_sd = jnp.concatenate([W1[0:D_NODE], W1[D_NODE:2 * D_NODE]], axis=1)
    b12 = 0.5 * jnp.concatenate([b1, b1])[None, :]
    w1et = W1[2 * D_NODE:2 * D_NODE + DE].T
    w1dt = W1[2 * D_NODE + DE:2 * D_NODE + 2 * DE].T
    w2t = W2.T
    b2c = jnp.tile(b2[:, None], (1, 128))
    gc = jnp.tile(ln_g[:, None], (1, 128))
    bc = jnp.tile(ln_b[:, None], (1, 128))

    psrc, pdst = _node_project(node_scalars, w_sd, b12)

    si3 = src_idxs.astype(jnp.int32).reshape(NCK, GPC, GC)
    di3 = dst_idxs.astype(jnp.int32).reshape(NCK, GPC, GC)
    s_a = _sc_gather(psrc, pdst, si3[:NCK_A], di3[:NCK_A])
    s_b = _sc_gather(psrc, pdst, si3[NCK_A:], di3[NCK_A:])

    out_t = _epilogue(s_a.T, s_b.T, edge_feats.T, d.T,
                      w1et, w1dt, w2t, b2c, gc, bc)
    return out_t.T
